# baseline (speedup 1.0000x reference)
import functools

import jax
import jax.numpy as jnp
from jax.experimental import pallas as pl
from jax.experimental.pallas import tpu as pltpu

_GK = 64
_NBUF = 4
_NCHUNK = 50


def _mlm_kernel(idx_ref, num_ref, x_hbm, loss_ref, num_out_ref, buf, acc, sems,
                *, vocab: int):
    i = pl.program_id(0)
    nsteps = pl.num_programs(0)
    nchunks = _NCHUNK
    chunk = vocab // _NCHUNK

    def issue(g, slot):
        for k in range(_GK):
            pltpu.make_async_copy(
                x_hbm.at[idx_ref[g * _GK + k]],
                buf.at[slot, k],
                sems.at[slot],
            ).start()

    @pl.when(i == 0)
    def _prologue():
        acc[...] = jnp.zeros((_GK, 1), jnp.float32)
        for g in range(_NBUF):
            @pl.when(g < nsteps)
            def _():
                issue(g, g)

    slot = jax.lax.rem(i, _NBUF)
    pltpu.make_async_copy(
        x_hbm.at[pl.ds(0, _GK)], buf.at[slot], sems.at[slot]
    ).wait()

    x = buf[slot]
    m = jnp.max(x, axis=1, keepdims=True)

    def p2(c, carry):
        dp, sp = carry
        d = buf[slot, :, pl.ds(c * chunk, chunk)] - m
        return dp + d, sp + jnp.exp(d)

    d0 = jnp.zeros((_GK, chunk), jnp.float32)
    s0 = jnp.zeros((_GK, chunk), jnp.float32)
    dp, sp = jax.lax.fori_loop(0, nchunks, p2, (d0, s0), unroll=True)
    td = jnp.sum(dp, axis=1, keepdims=True)
    s = jnp.sum(sp, axis=1, keepdims=True)

    j = i * _GK + jax.lax.broadcasted_iota(jnp.int32, (_GK, 1), 0)
    w = (j < num_ref[0]).astype(jnp.float32)
    acc[...] += w * (td - vocab * jnp.log(s))

    @pl.when(i + _NBUF < nsteps)
    def _refill():
        issue(i + _NBUF, slot)

    @pl.when(i == nsteps - 1)
    def _fin():
        numf = num_ref[0].astype(jnp.float32)
        loss_ref[0, 0] = -(jnp.sum(acc[...]) / (numf * vocab))
        num_out_ref[0, 0] = num_ref[0]


@jax.jit
def kernel(logits, labels):
    B, S, V = logits.shape
    R = B * S
    x = logits.reshape(R, V)
    mask = labels.reshape(R) != -100
    num = jnp.sum(mask.astype(jnp.int32))
    idx = jnp.nonzero(mask, size=R, fill_value=0)[0].astype(jnp.int32)
    num_steps = jnp.maximum((num + _GK - 1) // _GK, 1)

    grid_spec = pltpu.PrefetchScalarGridSpec(
        num_scalar_prefetch=2,
        grid=(num_steps,),
        in_specs=[pl.BlockSpec(memory_space=pl.ANY)],
        out_specs=[
            pl.BlockSpec(memory_space=pltpu.SMEM),
            pl.BlockSpec(memory_space=pltpu.SMEM),
        ],
        scratch_shapes=[
            pltpu.VMEM((_NBUF, _GK, V), jnp.float32),
            pltpu.VMEM((_GK, 1), jnp.float32),
            pltpu.SemaphoreType.DMA((_NBUF,)),
        ],
    )

    loss, num_out = pl.pallas_call(
        functools.partial(_mlm_kernel, vocab=V),
        grid_spec=grid_spec,
        out_shape=[
            jax.ShapeDtypeStruct((1, 1), jnp.float32),
            jax.ShapeDtypeStruct((1, 1), jnp.int32),
        ],
        compiler_params=pltpu.CompilerParams(
            dimension_semantics=("arbitrary",),
        ),
    )(idx, num.reshape(1), x)
    return (loss[0, 0], num_out[0, 0])
